# baseline (speedup 1.0000x reference)
import functools

import jax
import jax.numpy as jnp
from jax import lax
from jax.experimental import pallas as pl
from jax.experimental.pallas import tpu as pltpu
from jax.experimental.pallas import tpu_sc as plsc

N = 10000
NP = 10240
E = 160000
D = 128
C = 64

NC = 2
NS = 16
W = 128

W32 = 40
EP32 = NC * NS * W32 * W

WM = 64
NWM = 80

R = 1280
GRID = NP // R


def _mesh():
    return plsc.VectorSubcoreMesh(core_axis_name="c", subcore_axis_name="s")


_SC_PARAMS = pltpu.CompilerParams(needs_layout_passes=False)


@functools.partial(
    pl.kernel,
    mesh=_mesh(),
    compiler_params=_SC_PARAMS,
    out_type=jax.ShapeDtypeStruct((NC, NP, 16), jnp.float32),
    scratch_types=[
        pltpu.VMEM((W32, W), jnp.int32),
        pltpu.VMEM((W, 16), jnp.float32),
        pltpu.VMEM_SHARED((NP, 16), jnp.float32),
    ],
)
def _sc_degree(dst_hbm, out_hbm, dst_v, buf_v, acc_s):
    c = lax.axis_index("c")
    s = lax.axis_index("s")
    wid = s * NC + c
    pltpu.sync_copy(dst_hbm.at[wid], dst_v)

    def _zero_row(i, _):
        buf_v[i, :] = jnp.zeros((16,), jnp.float32)
        return 0

    lax.fori_loop(0, W, _zero_row, 0)
    base = s * (NP // NS)
    for k in range(NP // NS // W):
        pltpu.sync_copy(buf_v, acc_s.at[pl.ds(base + k * W, W)])
    plsc.subcore_barrier()

    def _one_row(i, _):
        buf_v[i, :] = jnp.ones((16,), jnp.float32)
        return 0

    lax.fori_loop(0, W, _one_row, 0)

    def _win(w, _):
        pltpu.sync_copy(buf_v, acc_s.at[dst_v.at[w]], add=True)
        return 0

    lax.fori_loop(0, W32, _win, 0)
    plsc.subcore_barrier()

    def _zero_row2(i, _):
        buf_v[i, :] = jnp.zeros((16,), jnp.float32)
        return 0

    lax.fori_loop(0, W, _zero_row2, 0)
    for k in range(NP // NS // W):
        pltpu.sync_copy(acc_s.at[pl.ds(base + k * W, W)], buf_v)
        pltpu.sync_copy(buf_v, out_hbm.at[c, pl.ds(base + k * W, W)])


@functools.partial(
    pl.kernel,
    mesh=_mesh(),
    compiler_params=_SC_PARAMS,
    out_type=jax.ShapeDtypeStruct((NC, NP, 16), jnp.float32),
    scratch_types=[
        pltpu.VMEM((W32, W), jnp.int32),
        pltpu.VMEM((W32, W), jnp.int32),
        pltpu.VMEM((2 * NP,), jnp.float32),
        pltpu.VMEM((W, 16), jnp.float32),
        pltpu.VMEM_SHARED((NP, 16), jnp.float32),
    ],
)
def _sc_ranking(src_hbm, dst_hbm, g_hbm, out_hbm,
                src_v, dst_v, g_v, buf_v, acc_s):
    c = lax.axis_index("c")
    s = lax.axis_index("s")
    wid = s * NC + c
    pltpu.sync_copy(src_hbm.at[wid], src_v)
    pltpu.sync_copy(dst_hbm.at[wid], dst_v)
    pltpu.sync_copy(g_hbm, g_v)

    def _zero_row(i, _):
        buf_v[i, :] = jnp.zeros((16,), jnp.float32)
        return 0

    lax.fori_loop(0, W, _zero_row, 0)
    base = s * (NP // NS)
    for k in range(NP // NS // W):
        pltpu.sync_copy(buf_v, acc_s.at[pl.ds(base + k * W, W)])
    plsc.subcore_barrier()

    lane = lax.iota(jnp.int32, 16)
    zero16 = jnp.zeros((16,), jnp.int32)
    one16 = jnp.ones((16,), jnp.int32)

    def _build(w, buf):
        for j in range(W // 16):
            idx = src_v[w, pl.ds(j * 16, 16)]
            i2 = idx * 2
            v0 = plsc.load_gather(g_v, [i2])
            v1 = plsc.load_gather(g_v, [i2 + one16])
            rows = lane + (j * 16)
            plsc.store_scatter(buf, [rows, zero16], v0)
            plsc.store_scatter(buf, [rows, one16], v1)

    def _win(w, _):
        _build(w, buf_v)
        pltpu.sync_copy(buf_v, acc_s.at[dst_v.at[w]], add=True)
        return 0

    lax.fori_loop(0, W32, _win, 0)
    plsc.subcore_barrier()

    def _zero_row2(i, _):
        buf_v[i, :] = jnp.zeros((16,), jnp.float32)
        return 0

    lax.fori_loop(0, W, _zero_row2, 0)
    for k in range(NP // NS // W):
        pltpu.sync_copy(acc_s.at[pl.ds(base + k * W, W)], buf_v)
        pltpu.sync_copy(buf_v, out_hbm.at[c, pl.ds(base + k * W, W)])


@functools.partial(
    pl.kernel,
    mesh=_mesh(),
    compiler_params=_SC_PARAMS,
    out_type=jax.ShapeDtypeStruct((NC, NP, D), jnp.float32),
    scratch_types=[
        pltpu.VMEM((NWM, WM), jnp.int32),
        pltpu.VMEM((NWM, WM), jnp.int32),
        pltpu.VMEM((WM, D), jnp.float32),
        pltpu.VMEM((WM, D), jnp.float32),
        pltpu.VMEM((WM, D), jnp.float32),
        pltpu.VMEM_SHARED((NP, D), jnp.float32),
        pltpu.SemaphoreType.DMA,
        pltpu.SemaphoreType.DMA,
        pltpu.SemaphoreType.DMA,
    ],
)
def _sc_main(src_hbm, dst_hbm, z_hbm, out_hbm,
             src_v, dst_v, r0, r1, r2, acc_s, m0, m1, m2):
    rows = (r0, r1, r2)
    sems = (m0, m1, m2)
    c = lax.axis_index("c")
    s = lax.axis_index("s")
    wid = s * NC + c
    pltpu.sync_copy(src_hbm.at[wid], src_v)
    pltpu.sync_copy(dst_hbm.at[wid], dst_v)

    def _zero_row(i, _):
        for q in range(D // 16):
            r0[i, pl.ds(q * 16, 16)] = jnp.zeros((16,), jnp.float32)
        return 0

    lax.fori_loop(0, WM, _zero_row, 0)
    base = s * (NP // NS)
    for k in range(NP // NS // WM):
        pltpu.sync_copy(r0, acc_s.at[pl.ds(base + k * WM, WM)])
    plsc.subcore_barrier()

    BLK = 8

    def _win(i, _):
        w0 = i * BLK
        cp = pltpu.async_copy(z_hbm.at[src_v.at[w0]], rows[0], sems[0])
        for k in range(BLK):
            cp.wait()
            if k + 1 < BLK:
                b = (k + 1) % 3
                cp = pltpu.async_copy(z_hbm.at[src_v.at[w0 + k + 1]],
                                      rows[b], sems[b])
            pltpu.sync_copy(rows[k % 3], acc_s.at[dst_v.at[w0 + k]], add=True)
        return 0

    lax.fori_loop(0, NWM // BLK, _win, 0)
    plsc.subcore_barrier()
    for k in range(NP // NS // WM):
        pltpu.sync_copy(acc_s.at[pl.ds(base + k * WM, WM)], r0)
        pltpu.sync_copy(r0, out_hbm.at[c, pl.ds(base + k * WM, WM)])


def _tc_proj_body(x_ref, w_ref, o_ref):
    o_ref[...] = jnp.dot(x_ref[...], w_ref[...],
                         preferred_element_type=jnp.float32)


def _tc_gate_body(deg_ref, xcr_ref, g_ref):
    d = deg_ref[0, :, 0:1] + deg_ref[1, :, 0:1] + 1.0
    dinv = lax.rsqrt(d)
    xr = xcr_ref[:, C:C + 2]
    g_ref[...] = dinv * xr


def _tc_z_body(deg_ref, racc_ref, xcr_ref, z_ref, pv_ref):
    d = deg_ref[0, :, 0:1] + deg_ref[1, :, 0:1] + 1.0
    dinv = lax.rsqrt(d)
    xr = xcr_ref[:, C:C + 2]
    r2 = racc_ref[0, :, 0:2] + racc_ref[1, :, 0:2]
    ranking = dinv * r2 + dinv * dinv * xr
    pv = 1.0 / (1.0 + jnp.exp(-ranking))
    q = pv * pv
    xc = xcr_ref[:, 0:C]
    z_ref[...] = jnp.concatenate([q[:, 0:1] * xc, q[:, 1:2] * xc], axis=1)
    pv_ref[...] = jnp.concatenate(
        [pv, jnp.zeros((R, 14), jnp.float32)], axis=1)


def _tc_out_body(s_ref, pv_ref, o_ref):
    s0 = s_ref[0, :, 0:C] + s_ref[1, :, 0:C]
    s1 = s_ref[0, :, C:D] + s_ref[1, :, C:D]
    comb = 0.5 * (pv_ref[:, 0:1] * s0 + pv_ref[:, 1:2] * s1)
    m = jnp.max(comb, axis=-1, keepdims=True)
    e = jnp.exp(comb - m)
    lse = jnp.log(jnp.sum(e, axis=-1, keepdims=True)) + m
    o_ref[...] = comb - lse


_tc_proj = pl.pallas_call(
    _tc_proj_body,
    grid=(GRID,),
    in_specs=[
        pl.BlockSpec((R, D), lambda i: (i, 0)),
        pl.BlockSpec((D, D), lambda i: (0, 0)),
    ],
    out_specs=pl.BlockSpec((R, D), lambda i: (i, 0)),
    out_shape=jax.ShapeDtypeStruct((NP, D), jnp.float32),
)

_tc_gate = pl.pallas_call(
    _tc_gate_body,
    grid=(GRID,),
    in_specs=[
        pl.BlockSpec((NC, R, 16), lambda i: (0, i, 0)),
        pl.BlockSpec((R, D), lambda i: (i, 0)),
    ],
    out_specs=pl.BlockSpec((R, 2), lambda i: (i, 0)),
    out_shape=jax.ShapeDtypeStruct((NP, 2), jnp.float32),
)

_tc_z = pl.pallas_call(
    _tc_z_body,
    grid=(GRID,),
    in_specs=[
        pl.BlockSpec((NC, R, 16), lambda i: (0, i, 0)),
        pl.BlockSpec((NC, R, 16), lambda i: (0, i, 0)),
        pl.BlockSpec((R, D), lambda i: (i, 0)),
    ],
    out_specs=[
        pl.BlockSpec((R, D), lambda i: (i, 0)),
        pl.BlockSpec((R, 16), lambda i: (i, 0)),
    ],
    out_shape=[
        jax.ShapeDtypeStruct((NP, D), jnp.float32),
        jax.ShapeDtypeStruct((NP, 16), jnp.float32),
    ],
)

_tc_out = pl.pallas_call(
    _tc_out_body,
    grid=(GRID,),
    in_specs=[
        pl.BlockSpec((NC, R, D), lambda i: (0, i, 0)),
        pl.BlockSpec((R, 16), lambda i: (i, 0)),
    ],
    out_specs=pl.BlockSpec((R, C), lambda i: (i, 0)),
    out_shape=jax.ShapeDtypeStruct((NP, C), jnp.float32),
)


def kernel(x, edge_index, W_rank, W_cocn):
    src = edge_index[0].astype(jnp.int32)
    dst = edge_index[1].astype(jnp.int32)

    p32 = EP32 - E
    junk = N + (jnp.arange(p32, dtype=jnp.int32) % (NP - N))
    src32 = jnp.concatenate([src, junk]).reshape(NC * NS, W32, W)
    dst32 = jnp.concatenate([dst, junk]).reshape(NC * NS, W32, W)

    x_pad = jnp.zeros((NP, D), jnp.float32).at[:N].set(x)
    W2 = jnp.zeros((D, D), jnp.float32)
    W2 = W2.at[:, :C].set(W_cocn).at[:, C:C + 2].set(W_rank)

    srcm = src32.reshape(NC * NS, NWM, WM)
    dstm = dst32.reshape(NC * NS, NWM, WM)

    xcr = _tc_proj(x_pad, W2)
    deg2 = _sc_degree(dst32)
    g2 = _tc_gate(deg2, xcr)
    gf = g2.reshape(2 * NP)
    racc2 = _sc_ranking(src32, dst32, gf)
    z, pv = _tc_z(deg2, racc2, xcr)
    s2 = _sc_main(srcm, dstm, z)
    out = _tc_out(s2, pv)
    return out[:N]


# speedup vs baseline: 1.0830x; 1.0830x over previous
import functools

import jax
import jax.numpy as jnp
from jax import lax
from jax.experimental import pallas as pl
from jax.experimental.pallas import tpu as pltpu
from jax.experimental.pallas import tpu_sc as plsc

N = 10000
NP = 10240
E = 160000
D = 128
C = 64

NC = 2
NS = 16
W = 128

W32 = 40
EP32 = NC * NS * W32 * W

WM = 128
NWM = 40

R = 1280
GRID = NP // R


def _mesh():
    return plsc.VectorSubcoreMesh(core_axis_name="c", subcore_axis_name="s")


_SC_PARAMS = pltpu.CompilerParams(needs_layout_passes=False)


@functools.partial(
    pl.kernel,
    mesh=_mesh(),
    compiler_params=_SC_PARAMS,
    out_type=jax.ShapeDtypeStruct((NC, NP, 16), jnp.float32),
    scratch_types=[
        pltpu.VMEM((W32, W), jnp.int32),
        pltpu.VMEM((W, 16), jnp.float32),
        pltpu.VMEM_SHARED((NP, 16), jnp.float32),
    ],
)
def _sc_degree(dst_hbm, out_hbm, dst_v, buf_v, acc_s):
    c = lax.axis_index("c")
    s = lax.axis_index("s")
    wid = s * NC + c
    pltpu.sync_copy(dst_hbm.at[wid], dst_v)

    def _zero_row(i, _):
        buf_v[i, :] = jnp.zeros((16,), jnp.float32)
        return 0

    lax.fori_loop(0, W, _zero_row, 0)
    base = s * (NP // NS)
    for k in range(NP // NS // W):
        pltpu.sync_copy(buf_v, acc_s.at[pl.ds(base + k * W, W)])
    plsc.subcore_barrier()

    def _one_row(i, _):
        buf_v[i, :] = jnp.ones((16,), jnp.float32)
        return 0

    lax.fori_loop(0, W, _one_row, 0)

    def _win(w, _):
        pltpu.sync_copy(buf_v, acc_s.at[dst_v.at[w]], add=True)
        return 0

    lax.fori_loop(0, W32, _win, 0)
    plsc.subcore_barrier()

    def _zero_row2(i, _):
        buf_v[i, :] = jnp.zeros((16,), jnp.float32)
        return 0

    lax.fori_loop(0, W, _zero_row2, 0)
    for k in range(NP // NS // W):
        pltpu.sync_copy(acc_s.at[pl.ds(base + k * W, W)], buf_v)
        pltpu.sync_copy(buf_v, out_hbm.at[c, pl.ds(base + k * W, W)])


@functools.partial(
    pl.kernel,
    mesh=_mesh(),
    compiler_params=_SC_PARAMS,
    out_type=jax.ShapeDtypeStruct((NC, NP, 16), jnp.float32),
    scratch_types=[
        pltpu.VMEM((W32, W), jnp.int32),
        pltpu.VMEM((W32, W), jnp.int32),
        pltpu.VMEM((2 * NP,), jnp.float32),
        pltpu.VMEM((W, 16), jnp.float32),
        pltpu.VMEM_SHARED((NP, 16), jnp.float32),
    ],
)
def _sc_ranking(src_hbm, dst_hbm, g_hbm, out_hbm,
                src_v, dst_v, g_v, buf_v, acc_s):
    c = lax.axis_index("c")
    s = lax.axis_index("s")
    wid = s * NC + c
    pltpu.sync_copy(src_hbm.at[wid], src_v)
    pltpu.sync_copy(dst_hbm.at[wid], dst_v)
    pltpu.sync_copy(g_hbm, g_v)

    def _zero_row(i, _):
        buf_v[i, :] = jnp.zeros((16,), jnp.float32)
        return 0

    lax.fori_loop(0, W, _zero_row, 0)
    base = s * (NP // NS)
    for k in range(NP // NS // W):
        pltpu.sync_copy(buf_v, acc_s.at[pl.ds(base + k * W, W)])
    plsc.subcore_barrier()

    lane = lax.iota(jnp.int32, 16)
    zero16 = jnp.zeros((16,), jnp.int32)
    one16 = jnp.ones((16,), jnp.int32)

    def _build(w, buf):
        for j in range(W // 16):
            idx = src_v[w, pl.ds(j * 16, 16)]
            i2 = idx * 2
            v0 = plsc.load_gather(g_v, [i2])
            v1 = plsc.load_gather(g_v, [i2 + one16])
            rows = lane + (j * 16)
            plsc.store_scatter(buf, [rows, zero16], v0)
            plsc.store_scatter(buf, [rows, one16], v1)

    def _win(w, _):
        _build(w, buf_v)
        pltpu.sync_copy(buf_v, acc_s.at[dst_v.at[w]], add=True)
        return 0

    lax.fori_loop(0, W32, _win, 0)
    plsc.subcore_barrier()

    def _zero_row2(i, _):
        buf_v[i, :] = jnp.zeros((16,), jnp.float32)
        return 0

    lax.fori_loop(0, W, _zero_row2, 0)
    for k in range(NP // NS // W):
        pltpu.sync_copy(acc_s.at[pl.ds(base + k * W, W)], buf_v)
        pltpu.sync_copy(buf_v, out_hbm.at[c, pl.ds(base + k * W, W)])


@functools.partial(
    pl.kernel,
    mesh=_mesh(),
    compiler_params=_SC_PARAMS,
    out_type=jax.ShapeDtypeStruct((NC, NP, D), jnp.float32),
    scratch_types=[
        pltpu.VMEM((NWM, WM), jnp.int32),
        pltpu.VMEM((NWM, WM), jnp.int32),
        pltpu.VMEM((WM, D), jnp.float32),
        pltpu.VMEM((WM, D), jnp.float32),
        pltpu.VMEM_SHARED((NP, D), jnp.float32),
        pltpu.SemaphoreType.DMA,
        pltpu.SemaphoreType.DMA,
    ],
)
def _sc_main(src_hbm, dst_hbm, z_hbm, out_hbm,
             src_v, dst_v, r0, r1, acc_s, m0, m1):
    rows = (r0, r1)
    sems = (m0, m1)
    c = lax.axis_index("c")
    s = lax.axis_index("s")
    wid = s * NC + c
    pltpu.sync_copy(src_hbm.at[wid], src_v)
    pltpu.sync_copy(dst_hbm.at[wid], dst_v)

    def _zero_row(i, _):
        for q in range(D // 16):
            r0[i, pl.ds(q * 16, 16)] = jnp.zeros((16,), jnp.float32)
        return 0

    lax.fori_loop(0, WM, _zero_row, 0)
    base = s * (NP // NS)
    for k in range(NP // NS // WM):
        pltpu.sync_copy(r0, acc_s.at[pl.ds(base + k * WM, WM)])
    plsc.subcore_barrier()

    def _win(i, _):
        w0 = i * 2
        w1 = w0 + 1
        cp0 = pltpu.async_copy(z_hbm.at[src_v.at[w0]], rows[0], sems[0])
        cp0.wait()
        cp1 = pltpu.async_copy(z_hbm.at[src_v.at[w1]], rows[1], sems[1])
        pltpu.sync_copy(rows[0], acc_s.at[dst_v.at[w0]], add=True)
        cp1.wait()
        pltpu.sync_copy(rows[1], acc_s.at[dst_v.at[w1]], add=True)
        return 0

    lax.fori_loop(0, NWM // 2, _win, 0)
    plsc.subcore_barrier()
    for k in range(NP // NS // WM):
        pltpu.sync_copy(acc_s.at[pl.ds(base + k * WM, WM)], r0)
        pltpu.sync_copy(r0, out_hbm.at[c, pl.ds(base + k * WM, WM)])


def _tc_proj_body(x_ref, w_ref, o_ref):
    o_ref[...] = jnp.dot(x_ref[...], w_ref[...],
                         preferred_element_type=jnp.float32)


def _tc_gate_body(deg_ref, xcr_ref, g_ref):
    d = deg_ref[0, :, 0:1] + deg_ref[1, :, 0:1] + 1.0
    dinv = lax.rsqrt(d)
    xr = xcr_ref[:, C:C + 2]
    g_ref[...] = dinv * xr


def _tc_z_body(deg_ref, racc_ref, xcr_ref, z_ref, pv_ref):
    d = deg_ref[0, :, 0:1] + deg_ref[1, :, 0:1] + 1.0
    dinv = lax.rsqrt(d)
    xr = xcr_ref[:, C:C + 2]
    r2 = racc_ref[0, :, 0:2] + racc_ref[1, :, 0:2]
    ranking = dinv * r2 + dinv * dinv * xr
    pv = 1.0 / (1.0 + jnp.exp(-ranking))
    q = pv * pv
    xc = xcr_ref[:, 0:C]
    z_ref[...] = jnp.concatenate([q[:, 0:1] * xc, q[:, 1:2] * xc], axis=1)
    pv_ref[...] = jnp.concatenate(
        [pv, jnp.zeros((R, 14), jnp.float32)], axis=1)


def _tc_out_body(s_ref, pv_ref, o_ref):
    s0 = s_ref[0, :, 0:C] + s_ref[1, :, 0:C]
    s1 = s_ref[0, :, C:D] + s_ref[1, :, C:D]
    comb = 0.5 * (pv_ref[:, 0:1] * s0 + pv_ref[:, 1:2] * s1)
    m = jnp.max(comb, axis=-1, keepdims=True)
    e = jnp.exp(comb - m)
    lse = jnp.log(jnp.sum(e, axis=-1, keepdims=True)) + m
    o_ref[...] = comb - lse


_tc_proj = pl.pallas_call(
    _tc_proj_body,
    grid=(GRID,),
    in_specs=[
        pl.BlockSpec((R, D), lambda i: (i, 0)),
        pl.BlockSpec((D, D), lambda i: (0, 0)),
    ],
    out_specs=pl.BlockSpec((R, D), lambda i: (i, 0)),
    out_shape=jax.ShapeDtypeStruct((NP, D), jnp.float32),
)

_tc_gate = pl.pallas_call(
    _tc_gate_body,
    grid=(GRID,),
    in_specs=[
        pl.BlockSpec((NC, R, 16), lambda i: (0, i, 0)),
        pl.BlockSpec((R, D), lambda i: (i, 0)),
    ],
    out_specs=pl.BlockSpec((R, 2), lambda i: (i, 0)),
    out_shape=jax.ShapeDtypeStruct((NP, 2), jnp.float32),
)

_tc_z = pl.pallas_call(
    _tc_z_body,
    grid=(GRID,),
    in_specs=[
        pl.BlockSpec((NC, R, 16), lambda i: (0, i, 0)),
        pl.BlockSpec((NC, R, 16), lambda i: (0, i, 0)),
        pl.BlockSpec((R, D), lambda i: (i, 0)),
    ],
    out_specs=[
        pl.BlockSpec((R, D), lambda i: (i, 0)),
        pl.BlockSpec((R, 16), lambda i: (i, 0)),
    ],
    out_shape=[
        jax.ShapeDtypeStruct((NP, D), jnp.float32),
        jax.ShapeDtypeStruct((NP, 16), jnp.float32),
    ],
)

_tc_out = pl.pallas_call(
    _tc_out_body,
    grid=(GRID,),
    in_specs=[
        pl.BlockSpec((NC, R, D), lambda i: (0, i, 0)),
        pl.BlockSpec((R, 16), lambda i: (i, 0)),
    ],
    out_specs=pl.BlockSpec((R, C), lambda i: (i, 0)),
    out_shape=jax.ShapeDtypeStruct((NP, C), jnp.float32),
)


def kernel(x, edge_index, W_rank, W_cocn):
    src = edge_index[0].astype(jnp.int32)
    dst = edge_index[1].astype(jnp.int32)

    p32 = EP32 - E
    junk = N + (jnp.arange(p32, dtype=jnp.int32) % (NP - N))
    src32 = jnp.concatenate([src, junk]).reshape(NC * NS, W32, W)
    dst32 = jnp.concatenate([dst, junk]).reshape(NC * NS, W32, W)

    x_pad = jnp.zeros((NP, D), jnp.float32).at[:N].set(x)
    W2 = jnp.zeros((D, D), jnp.float32)
    W2 = W2.at[:, :C].set(W_cocn).at[:, C:C + 2].set(W_rank)

    srcm = src32.reshape(NC * NS, NWM, WM)
    dstm = dst32.reshape(NC * NS, NWM, WM)

    xcr = _tc_proj(x_pad, W2)
    deg2 = _sc_degree(dst32)
    g2 = _tc_gate(deg2, xcr)
    gf = g2.reshape(2 * NP)
    racc2 = _sc_ranking(src32, dst32, gf)
    z, pv = _tc_z(deg2, racc2, xcr)
    s2 = _sc_main(srcm, dstm, z)
    out = _tc_out(s2, pv)
    return out[:N]
